# Initial kernel scaffold; baseline (speedup 1.0000x reference)
import jax
import jax.numpy as jnp
from jax.experimental import pallas as pl


def kernel(x, pos_embed):
    raise NotImplementedError("write your pallas kernel here")


# speedup vs baseline: 2.1231x; 2.1231x over previous
import functools

import jax
import jax.numpy as jnp
from jax import lax
from jax.experimental import pallas as pl
from jax.experimental.pallas import tpu as pltpu
from jax.experimental.pallas import tpu_sc as plsc


def _gather_rows(idx_flat, pos_embed, n_rows, d):
    info = plsc.get_sparse_core_info()
    nw = info.num_cores * info.num_subcores
    rows_per_w = n_rows // nw
    chunk = 64
    n_chunks = rows_per_w // chunk
    mesh = plsc.VectorSubcoreMesh(core_axis_name="c", subcore_axis_name="s")

    @functools.partial(
        pl.kernel,
        mesh=mesh,
        out_type=jax.ShapeDtypeStruct((n_rows, d), jnp.float32),
        scratch_types=[
            pltpu.VMEM((chunk,), jnp.int32),
            pltpu.VMEM((chunk, d), jnp.float32),
            pltpu.SemaphoreType.DMA,
        ],
    )
    def k(table_hbm, idx_hbm, out_hbm, idx_v, rows_v, sem):
        wid = lax.axis_index("s") * info.num_cores + lax.axis_index("c")
        base = wid * rows_per_w

        def body(i, carry):
            off = base + i * chunk
            pltpu.sync_copy(idx_hbm.at[pl.ds(off, chunk)], idx_v)
            pltpu.async_copy(table_hbm.at[idx_v], rows_v, sem).wait()
            pltpu.sync_copy(rows_v, out_hbm.at[pl.ds(off, chunk)])
            return carry

        lax.fori_loop(0, n_chunks, body, 0)

    return k(pos_embed, idx_flat)


def kernel(x, pos_embed):
    if x.ndim == 1:
        x = x[None, :]
    b, l = x.shape
    v, d = pos_embed.shape
    idx_flat = x.reshape(b * l).astype(jnp.int32)
    out = _gather_rows(idx_flat, pos_embed, b * l, d)
    return out.reshape(b, l, d)


# speedup vs baseline: 2.1423x; 1.0090x over previous
import functools

import jax
import jax.numpy as jnp
from jax import lax
from jax.experimental import pallas as pl
from jax.experimental.pallas import tpu as pltpu
from jax.experimental.pallas import tpu_sc as plsc


def _gather_rows(idx_flat, pos_embed, n_rows, d):
    info = plsc.get_sparse_core_info()
    nw = info.num_cores * info.num_subcores
    rows_per_w = n_rows // nw
    chunk = 32
    n_pairs = rows_per_w // (2 * chunk)
    mesh = plsc.VectorSubcoreMesh(core_axis_name="c", subcore_axis_name="s")

    @functools.partial(
        pl.kernel,
        mesh=mesh,
        out_type=jax.ShapeDtypeStruct((n_rows, d), jnp.float32),
        scratch_types=[
            pltpu.VMEM((2, chunk), jnp.int32),
            pltpu.VMEM((2, chunk, d), jnp.float32),
            pltpu.SemaphoreType.DMA,
            pltpu.SemaphoreType.DMA,
            pltpu.SemaphoreType.DMA,
        ],
    )
    def k(table_hbm, idx_hbm, out_hbm, idx2, rows2, gsem, ssem0, ssem1):
        wid = lax.axis_index("s") * info.num_cores + lax.axis_index("c")
        base = wid * rows_per_w
        ssems = (ssem0, ssem1)

        def pair(p, carry):
            for b in range(2):
                i = 2 * p + b
                off = base + i * chunk

                @pl.when(p > 0)
                def _wait_prev_scatter():
                    pltpu.make_async_copy(
                        rows2.at[b], out_hbm.at[pl.ds(off, chunk)], ssems[b]
                    ).wait()

                pltpu.sync_copy(idx_hbm.at[pl.ds(off, chunk)], idx2.at[b])
                pltpu.async_copy(table_hbm.at[idx2.at[b]], rows2.at[b], gsem).wait()
                pltpu.async_copy(rows2.at[b], out_hbm.at[pl.ds(off, chunk)], ssems[b])
            return carry

        lax.fori_loop(0, n_pairs, pair, 0)
        for b in range(2):
            pltpu.make_async_copy(
                rows2.at[b], out_hbm.at[pl.ds(base, chunk)], ssems[b]
            ).wait()

    return k(pos_embed, idx_flat)


def kernel(x, pos_embed):
    if x.ndim == 1:
        x = x[None, :]
    b, l = x.shape
    v, d = pos_embed.shape
    idx_flat = x.reshape(b * l).astype(jnp.int32)
    out = _gather_rows(idx_flat, pos_embed, b * l, d)
    return out.reshape(b, l, d)


# speedup vs baseline: 2.2548x; 1.0525x over previous
import functools

import jax
import jax.numpy as jnp
from jax import lax
from jax.experimental import pallas as pl
from jax.experimental.pallas import tpu as pltpu
from jax.experimental.pallas import tpu_sc as plsc


def _gather_rows(idx_flat, pos_embed, n_rows, d):
    info = plsc.get_sparse_core_info()
    nw = info.num_cores * info.num_subcores
    rows_per_w = n_rows // nw
    chunk = 32
    n_pairs = rows_per_w // (2 * chunk)
    mesh = plsc.VectorSubcoreMesh(core_axis_name="c", subcore_axis_name="s")

    @functools.partial(
        pl.kernel,
        mesh=mesh,
        out_type=jax.ShapeDtypeStruct((n_rows, d), jnp.float32),
        scratch_types=[
            pltpu.VMEM((rows_per_w,), jnp.int32),
            pltpu.VMEM((2, chunk, d), jnp.float32),
            pltpu.SemaphoreType.DMA,
            pltpu.SemaphoreType.DMA,
            pltpu.SemaphoreType.DMA,
            pltpu.SemaphoreType.DMA,
        ],
    )
    def k(table_hbm, idx_hbm, out_hbm, idx_all, rows2, g0, g1, s0, s1):
        wid = lax.axis_index("s") * info.num_cores + lax.axis_index("c")
        base = wid * rows_per_w
        gsems = (g0, g1)
        ssems = (s0, s1)

        pltpu.sync_copy(idx_hbm.at[pl.ds(base, rows_per_w)], idx_all)

        def pair(p, carry):
            for b in range(2):
                i = 2 * p + b

                @pl.when(p > 0)
                def _wait_prev_scatter():
                    pltpu.make_async_copy(
                        rows2.at[b],
                        out_hbm.at[pl.ds(base + i * chunk, chunk)],
                        ssems[b],
                    ).wait()

                pltpu.async_copy(
                    table_hbm.at[idx_all.at[pl.ds(i * chunk, chunk)]],
                    rows2.at[b],
                    gsems[b],
                )
            for b in range(2):
                i = 2 * p + b
                off = base + i * chunk
                pltpu.make_async_copy(
                    table_hbm.at[idx_all.at[pl.ds(i * chunk, chunk)]],
                    rows2.at[b],
                    gsems[b],
                ).wait()
                pltpu.async_copy(rows2.at[b], out_hbm.at[pl.ds(off, chunk)], ssems[b])
            return carry

        lax.fori_loop(0, n_pairs, pair, 0)
        for b in range(2):
            pltpu.make_async_copy(
                rows2.at[b], out_hbm.at[pl.ds(base, chunk)], ssems[b]
            ).wait()

    return k(pos_embed, idx_flat)


def kernel(x, pos_embed):
    if x.ndim == 1:
        x = x[None, :]
    b, l = x.shape
    v, d = pos_embed.shape
    idx_flat = x.reshape(b * l).astype(jnp.int32)
    out = _gather_rows(idx_flat, pos_embed, b * l, d)
    return out.reshape(b, l, d)


# speedup vs baseline: 2.3321x; 1.0343x over previous
import functools

import jax
import jax.numpy as jnp
from jax import lax
from jax.experimental import pallas as pl
from jax.experimental.pallas import tpu as pltpu
from jax.experimental.pallas import tpu_sc as plsc


def _gather_rows(idx_flat, pos_embed, n_rows, d):
    info = plsc.get_sparse_core_info()
    nw = info.num_cores * info.num_subcores
    rows_per_w = n_rows // nw
    chunk = 16
    nbuf = 4
    n_groups = rows_per_w // (nbuf * chunk)
    mesh = plsc.VectorSubcoreMesh(core_axis_name="c", subcore_axis_name="s")

    @functools.partial(
        pl.kernel,
        mesh=mesh,
        out_type=jax.ShapeDtypeStruct((n_rows, d), jnp.float32),
        scratch_types=[
            pltpu.VMEM((rows_per_w,), jnp.int32),
            pltpu.VMEM((nbuf, chunk, d), jnp.float32),
        ]
        + [pltpu.SemaphoreType.DMA] * (2 * nbuf),
    )
    def k(table_hbm, idx_hbm, out_hbm, idx_all, rows, *sems):
        wid = lax.axis_index("s") * info.num_cores + lax.axis_index("c")
        base = wid * rows_per_w
        gsems = sems[:nbuf]
        ssems = sems[nbuf:]

        pltpu.sync_copy(idx_hbm.at[pl.ds(base, rows_per_w)], idx_all)

        def group(g, carry):
            for b in range(nbuf):
                i = nbuf * g + b

                @pl.when(g > 0)
                def _wait_prev_scatter():
                    pltpu.make_async_copy(
                        rows.at[b],
                        out_hbm.at[pl.ds(base + i * chunk, chunk)],
                        ssems[b],
                    ).wait()

                pltpu.async_copy(
                    table_hbm.at[idx_all.at[pl.ds(i * chunk, chunk)]],
                    rows.at[b],
                    gsems[b],
                )
            for b in range(nbuf):
                i = nbuf * g + b
                off = base + i * chunk
                pltpu.make_async_copy(
                    table_hbm.at[idx_all.at[pl.ds(i * chunk, chunk)]],
                    rows.at[b],
                    gsems[b],
                ).wait()
                pltpu.async_copy(rows.at[b], out_hbm.at[pl.ds(off, chunk)], ssems[b])
            return carry

        lax.fori_loop(0, n_groups, group, 0)
        for b in range(nbuf):
            pltpu.make_async_copy(
                rows.at[b], out_hbm.at[pl.ds(base, chunk)], ssems[b]
            ).wait()

    return k(pos_embed, idx_flat)


def kernel(x, pos_embed):
    if x.ndim == 1:
        x = x[None, :]
    b, l = x.shape
    v, d = pos_embed.shape
    idx_flat = x.reshape(b * l).astype(jnp.int32)
    out = _gather_rows(idx_flat, pos_embed, b * l, d)
    return out.reshape(b, l, d)
